# Initial kernel scaffold; baseline (speedup 1.0000x reference)
import jax
import jax.numpy as jnp
from jax.experimental import pallas as pl


def kernel(x_author, x_paper, edge_index_writes, edge_index_rev, W_kqv_author, b_kqv_author, W_kqv_paper, b_kqv_paper, W_k_rel, W_v_rel, W_out_author, b_out_author, W_out_paper, b_out_paper, skip_author, skip_paper, p_rel_writes, p_rel_rev):
    raise NotImplementedError("write your pallas kernel here")


# speedup vs baseline: 2.4236x; 2.4236x over previous
import functools
import math

import jax
import jax.numpy as jnp
from jax import lax
from jax.experimental import pallas as pl
from jax.experimental.pallas import tpu as pltpu
from jax.experimental.pallas import tpu_sc as plsc

N_NODE = 10000
E_PER = 160000
D_IN = 256
D_OUT = 256
H = 8
D = 32

NTILE = 16
TPT = E_PER // NTILE
CA = 80
CB = 80
N_PAD = 10240
RPT = N_PAD // NTILE


def _qkv_body(x_ref, w_ref, b_ref, q_ref, k_ref, v0_ref, v1_ref):
    acc = jnp.dot(x_ref[...], w_ref[...], preferred_element_type=jnp.float32)
    acc = acc + b_ref[...]
    q_ref[...] = acc[:, 0:256]
    k_ref[...] = acc[:, 256:512]
    v0_ref[...] = acc[:, 512:640]
    v1_ref[...] = acc[:, 640:768]


def _qkv(x, w, b):
    nb = 10
    rb = N_NODE // nb
    return pl.pallas_call(
        _qkv_body,
        grid=(nb,),
        in_specs=[
            pl.BlockSpec((rb, D_IN), lambda i: (i, 0)),
            pl.BlockSpec((D_IN, 3 * D_OUT), lambda i: (0, 0)),
            pl.BlockSpec((1, 3 * D_OUT), lambda i: (0, 0)),
        ],
        out_specs=[
            pl.BlockSpec((rb, D_OUT), lambda i: (i, 0)),
            pl.BlockSpec((rb, D_OUT), lambda i: (i, 0)),
            pl.BlockSpec((rb, 128), lambda i: (i, 0)),
            pl.BlockSpec((rb, 128), lambda i: (i, 0)),
        ],
        out_shape=[
            jax.ShapeDtypeStruct((N_NODE, D_OUT), jnp.float32),
            jax.ShapeDtypeStruct((N_NODE, D_OUT), jnp.float32),
            jax.ShapeDtypeStruct((N_NODE, 128), jnp.float32),
            jax.ShapeDtypeStruct((N_NODE, 128), jnp.float32),
        ],
    )(x, w, b)


@functools.lru_cache(maxsize=1)
def _sc_kernels():
  mesh = plsc.VectorSubcoreMesh(core_axis_name="c", subcore_axis_name="s")

  @functools.partial(
      pl.kernel,
      out_type=(
          jax.ShapeDtypeStruct((2 * E_PER, 16), jnp.float32),
          jax.ShapeDtypeStruct((2, N_PAD, 16), jnp.float32),
      ),
      mesh=mesh,
      compiler_params=pltpu.CompilerParams(use_tc_tiling_on_sc=False, needs_layout_passes=False),
      scratch_types=[
          pltpu.VMEM((CA,), jnp.int32),
          pltpu.VMEM((CA,), jnp.int32),
          pltpu.VMEM((CA,), jnp.int32),
          pltpu.VMEM((CA, 256), jnp.float32),
          pltpu.VMEM((CA, 256), jnp.float32),
          pltpu.VMEM((CA, 16), jnp.float32),
          pltpu.VMEM((RPT, 16), jnp.float32),
          pltpu.VMEM_SHARED((N_PAD, 16), jnp.float32),
          pltpu.SemaphoreType.DMA,
          pltpu.SemaphoreType.DMA,
      ],
  )
  def edge_w(src_hbm, dst_hbm, q_hbm, k_hbm, w_hbm, den_hbm,
             src_v, dst_v, dsto_v, q_rows, k_rows, w_stage, zbuf,
             den_sh, sem1, sem2):
    c = lax.axis_index("c")
    s = lax.axis_index("s")

    def zrow(j, carry):
      zbuf[j, :] = jnp.zeros((16,), jnp.float32)
      return carry
    lax.fori_loop(0, RPT, zrow, 0)
    pltpu.sync_copy(zbuf, den_sh.at[pl.ds(s * RPT, RPT)])
    plsc.subcore_barrier()
    lane = lax.iota(jnp.int32, 16)

    def chunk(g, carry):
      base = c * E_PER + s * TPT + g * CA
      pltpu.sync_copy(src_hbm.at[pl.ds(base, CA)], src_v)
      pltpu.sync_copy(dst_hbm.at[pl.ds(base, CA)], dst_v)
      off = c * N_NODE
      for i in range(CA // 16):
        sl = pl.ds(i * 16, 16)
        src_v[sl] = src_v[sl] + off
        dsto_v[sl] = dst_v[sl] + off
      cp1 = pltpu.async_copy(k_hbm.at[src_v], k_rows, sem1)
      cp2 = pltpu.async_copy(q_hbm.at[dsto_v], q_rows, sem2)
      cp1.wait()
      cp2.wait()

      def grp(t, carry2):
        rows = lane + t * 16
        for h in range(H):
          acc = jnp.zeros((16,), jnp.float32)
          for d in range(D):
            col = jnp.full((16,), h * D + d, jnp.int32)
            qd = plsc.load_gather(q_rows, [rows, col])
            kd = plsc.load_gather(k_rows, [rows, col])
            acc = acc + qd * kd
          plsc.store_scatter(w_stage, [rows, jnp.full((16,), h, jnp.int32)],
                             jnp.exp(acc))
        return carry2
      lax.fori_loop(0, CA // 16, grp, 0)

      pltpu.sync_copy(w_stage, w_hbm.at[pl.ds(base, CA)])
      pltpu.sync_copy(w_stage, den_sh.at[dst_v], add=True)
      return carry
    lax.fori_loop(0, TPT // CA, chunk, 0)

    plsc.subcore_barrier()
    pltpu.sync_copy(den_sh.at[pl.ds(s * RPT, RPT)],
                    den_hbm.at[c, pl.ds(s * RPT, RPT)])

  @functools.partial(
      pl.kernel,
      out_type=jax.ShapeDtypeStruct((2, 2, N_PAD, 128), jnp.float32),
      mesh=mesh,
      compiler_params=pltpu.CompilerParams(use_tc_tiling_on_sc=False, needs_layout_passes=False),
      scratch_types=[
          pltpu.VMEM((CB,), jnp.int32),
          pltpu.VMEM((CB,), jnp.int32),
          pltpu.VMEM((CB, 128), jnp.float32),
          pltpu.VMEM((CB, 128), jnp.float32),
          pltpu.VMEM((CB, 16), jnp.float32),
          pltpu.VMEM((128, 128), jnp.float32),
          pltpu.VMEM_SHARED((N_PAD, 128), jnp.float32),
          pltpu.SemaphoreType.DMA,
      ],
  )
  def edge_agg(src_hbm, dst_hbm, v0_hbm, v1_hbm, w_hbm, out_hbm,
               src_v, dst_v, v_rows, wv_stage, w_stage, zbuf, acc_sh, sem):
    c = lax.axis_index("c")
    s = lax.axis_index("s")

    def zrow(j, carry):
      for i in range(8):
        zbuf[j, pl.ds(i * 16, 16)] = jnp.zeros((16,), jnp.float32)
      return carry
    lax.fori_loop(0, 128, zrow, 0)

    for half in range(2):
      for t in range(5):
        pltpu.sync_copy(zbuf, acc_sh.at[pl.ds(s * RPT + t * 128, 128)])
      plsc.subcore_barrier()

      vtab = v0_hbm if half == 0 else v1_hbm

      def chunk(g, carry):
        base = c * E_PER + s * TPT + g * CB
        pltpu.sync_copy(src_hbm.at[pl.ds(base, CB)], src_v)
        pltpu.sync_copy(dst_hbm.at[pl.ds(base, CB)], dst_v)
        pltpu.sync_copy(w_hbm.at[pl.ds(base, CB)], w_stage)
        off = c * N_NODE
        for i in range(CB // 16):
          sl = pl.ds(i * 16, 16)
          src_v[sl] = src_v[sl] + off
        pltpu.async_copy(vtab.at[src_v], v_rows, sem).wait()

        def edge(e, carry2):
          erow = jnp.full((16,), e, jnp.int32)
          for hh in range(4):
            lane = jnp.full((16,), half * 4 + hh, jnp.int32)
            wspl = plsc.load_gather(w_stage, [erow, lane])
            for j in range(2):
              sl = pl.ds(hh * 32 + j * 16, 16)
              wv_stage[e, sl] = v_rows[e, sl] * wspl
          return carry2
        lax.fori_loop(0, CB, edge, 0)

        pltpu.sync_copy(wv_stage, acc_sh.at[dst_v], add=True)
        return carry
      lax.fori_loop(0, TPT // CB, chunk, 0)

      plsc.subcore_barrier()
      pltpu.sync_copy(acc_sh.at[pl.ds(s * RPT, RPT)],
                      out_hbm.at[c, half, pl.ds(s * RPT, RPT)])
      plsc.subcore_barrier()

  return edge_w, edge_agg


def _edge_w(src_cat, dst_cat, q_cat, k_cat):
  return _sc_kernels()[0](src_cat, dst_cat, q_cat, k_cat)


def _edge_agg(src_cat, dst_cat, v0_cat, v1_cat, w_e):
  return _sc_kernels()[1](src_cat, dst_cat, v0_cat, v1_cat, w_e)


def _out_body(wv0_ref, wv1_ref, den_ref, x_ref, w_ref, b_ref, s_ref, o_ref):
    wv = jnp.concatenate([wv0_ref[0, 0], wv1_ref[0, 0]], axis=1)
    den = den_ref[0]
    parts = []
    for h in range(H):
        parts.append(wv[:, h * 32:(h + 1) * 32] / (den[:, h:h + 1] + 1e-16))
    agg = jnp.concatenate(parts, axis=1)
    g = 0.5 * agg * (1.0 + lax.erf(agg * (1.0 / math.sqrt(2.0))))
    y = jnp.dot(g, w_ref[...], preferred_element_type=jnp.float32) + b_ref[...]
    a = s_ref[0, 0]
    o_ref[...] = a * y + (1.0 - a) * x_ref[...]


def _epilogue(wv, den, etype, x, w_out, b_out, sig):
    nb = 10
    rb = N_NODE // nb
    return pl.pallas_call(
        _out_body,
        grid=(nb,),
        in_specs=[
            pl.BlockSpec((1, 1, rb, 128), lambda i: (etype, 0, i, 0)),
            pl.BlockSpec((1, 1, rb, 128), lambda i: (etype, 1, i, 0)),
            pl.BlockSpec((1, rb, 16), lambda i: (etype, i, 0)),
            pl.BlockSpec((rb, D_IN), lambda i: (i, 0)),
            pl.BlockSpec((D_OUT, D_OUT), lambda i: (0, 0)),
            pl.BlockSpec((1, D_OUT), lambda i: (0, 0)),
            pl.BlockSpec((1, 1), lambda i: (0, 0)),
        ],
        out_specs=pl.BlockSpec((rb, D_OUT), lambda i: (i, 0)),
        out_shape=jax.ShapeDtypeStruct((N_NODE, D_OUT), jnp.float32),
    )(wv, wv, den, x, w_out, b_out, sig)


def kernel(x_author, x_paper, edge_index_writes, edge_index_rev,
           W_kqv_author, b_kqv_author, W_kqv_paper, b_kqv_paper,
           W_k_rel, W_v_rel,
           W_out_author, b_out_author, W_out_paper, b_out_paper,
           skip_author, skip_paper, p_rel_writes, p_rel_rev):
    f32 = jnp.float32
    scale = 1.0 / math.sqrt(D)
    hidx = jnp.arange(H) * 2

    def fold(W_kqv, b_kqv, et, p_rel):
        Wk = W_kqv[:, 0:256].reshape(D_IN, H, D)
        Wq = W_kqv[:, 256:512].reshape(D_IN, H, D)
        Wv = W_kqv[:, 512:768].reshape(D_IN, H, D)
        bk = b_kqv[0:256].reshape(H, D)
        bq = b_kqv[256:512].reshape(H, D)
        bv = b_kqv[512:768].reshape(H, D)
        Rk = W_k_rel[hidx + et]
        Rv = W_v_rel[hidx + et]
        qs = (p_rel[0] * scale)[None, :, None]
        Wq2 = (Wq * qs).reshape(D_IN, 256)
        bq2 = (bq * qs[0]).reshape(256)
        Wk2 = jnp.einsum('ihd,hdo->iho', Wk, Rk).reshape(D_IN, 256)
        bk2 = jnp.einsum('hd,hdo->ho', bk, Rk).reshape(256)
        Wv2 = jnp.einsum('ihd,hdo->iho', Wv, Rv).reshape(D_IN, 256)
        bv2 = jnp.einsum('hd,hdo->ho', bv, Rv).reshape(256)
        W = jnp.concatenate([Wq2, Wk2, Wv2], axis=1)
        b = jnp.concatenate([bq2, bk2, bv2])[None, :]
        return W, b

    Wa, ba = fold(W_kqv_author, b_kqv_author, 0, p_rel_rev)
    Wp, bp = fold(W_kqv_paper, b_kqv_paper, 1, p_rel_writes)

    qa, ka, va0, va1 = _qkv(x_author, Wa, ba)
    qp, kp, vp0, vp1 = _qkv(x_paper, Wp, bp)

    q_cat = jnp.concatenate([qp, qa], axis=0)
    k_cat = jnp.concatenate([ka, kp], axis=0)
    v0_cat = jnp.concatenate([va0, vp0], axis=0)
    v1_cat = jnp.concatenate([va1, vp1], axis=0)
    src_cat = jnp.concatenate([edge_index_writes[0], edge_index_rev[0]])
    dst_cat = jnp.concatenate([edge_index_writes[1], edge_index_rev[1]])

    w_e, den = _edge_w(src_cat, dst_cat, q_cat, k_cat)
    wv = _edge_agg(src_cat, dst_cat, v0_cat, v1_cat, w_e)

    sig_a = jax.nn.sigmoid(skip_author)[0].reshape(1, 1)
    sig_p = jax.nn.sigmoid(skip_paper)[0].reshape(1, 1)
    out_a = _epilogue(wv, den, 1, x_author, W_out_author,
                      b_out_author[None, :], sig_a)
    out_p = _epilogue(wv, den, 0, x_paper, W_out_paper,
                      b_out_paper[None, :], sig_p)
    return out_a, out_p


# speedup vs baseline: 4.4572x; 1.8391x over previous
import functools
import math

import jax
import jax.numpy as jnp
from jax import lax
from jax.experimental import pallas as pl
from jax.experimental.pallas import tpu as pltpu
from jax.experimental.pallas import tpu_sc as plsc

N_NODE = 10000
E_PER = 160000
D_IN = 256
D_OUT = 256
H = 8
D = 32

NTILE = 16
TPT = E_PER // NTILE
CA = 80
CB = 80
N_PAD = 10240
RPT = N_PAD // NTILE


def _qkv_body(x_ref, w_ref, b_ref, q_ref, k_ref, v0_ref, v1_ref):
    acc = jnp.dot(x_ref[...], w_ref[...], preferred_element_type=jnp.float32)
    acc = acc + b_ref[...]
    q_ref[...] = acc[:, 0:256]
    k_ref[...] = acc[:, 256:512]
    v0_ref[...] = acc[:, 512:640]
    v1_ref[...] = acc[:, 640:768]


def _qkv(x, w, b):
    nb = 10
    rb = N_NODE // nb
    return pl.pallas_call(
        _qkv_body,
        grid=(nb,),
        in_specs=[
            pl.BlockSpec((rb, D_IN), lambda i: (i, 0)),
            pl.BlockSpec((D_IN, 3 * D_OUT), lambda i: (0, 0)),
            pl.BlockSpec((1, 3 * D_OUT), lambda i: (0, 0)),
        ],
        out_specs=[
            pl.BlockSpec((rb, D_OUT), lambda i: (i, 0)),
            pl.BlockSpec((rb, D_OUT), lambda i: (i, 0)),
            pl.BlockSpec((rb, 128), lambda i: (i, 0)),
            pl.BlockSpec((rb, 128), lambda i: (i, 0)),
        ],
        out_shape=[
            jax.ShapeDtypeStruct((N_NODE, D_OUT), jnp.float32),
            jax.ShapeDtypeStruct((N_NODE, D_OUT), jnp.float32),
            jax.ShapeDtypeStruct((N_NODE, 128), jnp.float32),
            jax.ShapeDtypeStruct((N_NODE, 128), jnp.float32),
        ],
    )(x, w, b)


@functools.lru_cache(maxsize=1)
def _sc_kernels():
  mesh = plsc.VectorSubcoreMesh(core_axis_name="c", subcore_axis_name="s")
  params = pltpu.CompilerParams(use_tc_tiling_on_sc=False,
                                needs_layout_passes=False)

  @functools.partial(
      pl.kernel,
      out_type=(
          jax.ShapeDtypeStruct((2 * E_PER, 16), jnp.float32),
          jax.ShapeDtypeStruct((2, N_PAD, 16), jnp.float32),
      ),
      mesh=mesh,
      compiler_params=params,
      scratch_types=[
          pltpu.VMEM((CA,), jnp.int32),
          pltpu.VMEM((CA,), jnp.int32),
          pltpu.VMEM((CA,), jnp.int32),
          pltpu.VMEM((CA, 256), jnp.float32),
          pltpu.VMEM((CA, 256), jnp.float32),
          pltpu.VMEM((CA, 16), jnp.float32),
          pltpu.VMEM((RPT, 16), jnp.float32),
          pltpu.VMEM_SHARED((N_PAD, 16), jnp.float32),
          pltpu.SemaphoreType.DMA,
          pltpu.SemaphoreType.DMA,
      ],
  )
  def edge_w(src_hbm, dst_hbm, q_hbm, k_hbm, w_hbm, den_hbm,
             src_v, dst_v, dsto_v, q_rows, k_rows, w_stage, zbuf,
             den_sh, sem1, sem2):
    c = lax.axis_index("c")
    s = lax.axis_index("s")
    lane = lax.iota(jnp.int32, 16)
    z16 = jnp.zeros((16,), jnp.float32)

    def zrow(j, carry):
      zbuf[j, :] = z16
      return carry
    lax.fori_loop(0, RPT, zrow, 0)
    pltpu.sync_copy(zbuf, den_sh.at[pl.ds(s * RPT, RPT)])
    plsc.subcore_barrier()

    def chunk(g, carry):
      base = c * E_PER + s * TPT + g * CA
      pltpu.sync_copy(src_hbm.at[pl.ds(base, CA)], src_v)
      pltpu.sync_copy(dst_hbm.at[pl.ds(base, CA)], dst_v)
      off = c * N_NODE
      for i in range(CA // 16):
        sl = pl.ds(i * 16, 16)
        src_v[sl] = src_v[sl] + off
        dsto_v[sl] = dst_v[sl] + off
      cp1 = pltpu.async_copy(k_hbm.at[src_v], k_rows, sem1)
      cp2 = pltpu.async_copy(q_hbm.at[dsto_v], q_rows, sem2)
      cp1.wait()
      cp2.wait()

      def grp(t, carry2):
        rows = lane + t * 16
        for h in range(H):
          acc = z16
          for j in range(D):
            col = ((lane + j) & (D - 1)) + h * D
            qd = plsc.load_gather(q_rows, [rows, col])
            kd = plsc.load_gather(k_rows, [rows, col])
            acc = acc + qd * kd
          plsc.store_scatter(w_stage, [rows, jnp.full((16,), h, jnp.int32)],
                             jnp.exp(acc))
        return carry2
      lax.fori_loop(0, CA // 16, grp, 0)

      pltpu.sync_copy(w_stage, w_hbm.at[pl.ds(base, CA)])
      pltpu.sync_copy(w_stage, den_sh.at[dst_v], add=True)
      return carry
    lax.fori_loop(0, TPT // CA, chunk, 0)

    plsc.subcore_barrier()
    pltpu.sync_copy(den_sh.at[pl.ds(s * RPT, RPT)],
                    den_hbm.at[c, pl.ds(s * RPT, RPT)])

  @functools.partial(
      pl.kernel,
      out_type=jax.ShapeDtypeStruct((2, 2, N_PAD, 128), jnp.float32),
      mesh=mesh,
      compiler_params=params,
      scratch_types=[
          pltpu.VMEM((CB,), jnp.int32),
          pltpu.VMEM((CB,), jnp.int32),
          pltpu.VMEM((CB, 128), jnp.float32),
          pltpu.VMEM((CB, 128), jnp.float32),
          pltpu.VMEM((CB, 16), jnp.float32),
          pltpu.VMEM((128, 128), jnp.float32),
          pltpu.VMEM_SHARED((N_PAD, 128), jnp.float32),
          pltpu.SemaphoreType.DMA,
      ],
  )
  def edge_agg(src_hbm, dst_hbm, v0_hbm, v1_hbm, w_hbm, out_hbm,
               src_v, dst_v, v_rows, wv_stage, w_stage, zbuf, acc_sh, sem):
    c = lax.axis_index("c")
    s = lax.axis_index("s")
    lane = lax.iota(jnp.int32, 16)
    z16 = jnp.zeros((16,), jnp.float32)

    def zrow(j, carry):
      for i in range(8):
        zbuf[j, pl.ds(i * 16, 16)] = z16
      return carry
    lax.fori_loop(0, 128, zrow, 0)

    for half in range(2):
      for t in range(5):
        pltpu.sync_copy(zbuf, acc_sh.at[pl.ds(s * RPT + t * 128, 128)])
      plsc.subcore_barrier()

      vtab = v0_hbm if half == 0 else v1_hbm

      def chunk(g, carry):
        base = c * E_PER + s * TPT + g * CB
        pltpu.sync_copy(src_hbm.at[pl.ds(base, CB)], src_v)
        pltpu.sync_copy(dst_hbm.at[pl.ds(base, CB)], dst_v)
        pltpu.sync_copy(w_hbm.at[pl.ds(base, CB)], w_stage)
        off = c * N_NODE
        for i in range(CB // 16):
          sl = pl.ds(i * 16, 16)
          src_v[sl] = src_v[sl] + off
        pltpu.async_copy(vtab.at[src_v], v_rows, sem).wait()

        def grp(t, carry2):
          rows = lane + t * 16
          for hh in range(4):
            hcol = jnp.full((16,), half * 4 + hh, jnp.int32)
            wh = plsc.load_gather(w_stage, [rows, hcol])
            for j in range(D):
              col = ((lane + j) & (D - 1)) + hh * D
              vd = plsc.load_gather(v_rows, [rows, col])
              plsc.store_scatter(wv_stage, [rows, col], vd * wh)
          return carry2
        lax.fori_loop(0, CB // 16, grp, 0)

        pltpu.sync_copy(wv_stage, acc_sh.at[dst_v], add=True)
        return carry
      lax.fori_loop(0, TPT // CB, chunk, 0)

      plsc.subcore_barrier()
      pltpu.sync_copy(acc_sh.at[pl.ds(s * RPT, RPT)],
                      out_hbm.at[c, half, pl.ds(s * RPT, RPT)])
      plsc.subcore_barrier()

  return edge_w, edge_agg


def _edge_w(src_cat, dst_cat, q_cat, k_cat):
  return _sc_kernels()[0](src_cat, dst_cat, q_cat, k_cat)


def _edge_agg(src_cat, dst_cat, v0_cat, v1_cat, w_e):
  return _sc_kernels()[1](src_cat, dst_cat, v0_cat, v1_cat, w_e)


def _out_body(wv0_ref, wv1_ref, den_ref, x_ref, w_ref, b_ref, s_ref, o_ref):
    wv = jnp.concatenate([wv0_ref[0, 0], wv1_ref[0, 0]], axis=1)
    den = den_ref[0]
    parts = []
    for h in range(H):
        parts.append(wv[:, h * 32:(h + 1) * 32] / (den[:, h:h + 1] + 1e-16))
    agg = jnp.concatenate(parts, axis=1)
    g = 0.5 * agg * (1.0 + lax.erf(agg * (1.0 / math.sqrt(2.0))))
    y = jnp.dot(g, w_ref[...], preferred_element_type=jnp.float32) + b_ref[...]
    a = s_ref[0, 0]
    o_ref[...] = a * y + (1.0 - a) * x_ref[...]


def _epilogue(wv, den, etype, x, w_out, b_out, sig):
    nb = 10
    rb = N_NODE // nb
    return pl.pallas_call(
        _out_body,
        grid=(nb,),
        in_specs=[
            pl.BlockSpec((1, 1, rb, 128), lambda i: (etype, 0, i, 0)),
            pl.BlockSpec((1, 1, rb, 128), lambda i: (etype, 1, i, 0)),
            pl.BlockSpec((1, rb, 16), lambda i: (etype, i, 0)),
            pl.BlockSpec((rb, D_IN), lambda i: (i, 0)),
            pl.BlockSpec((D_OUT, D_OUT), lambda i: (0, 0)),
            pl.BlockSpec((1, D_OUT), lambda i: (0, 0)),
            pl.BlockSpec((1, 1), lambda i: (0, 0)),
        ],
        out_specs=pl.BlockSpec((rb, D_OUT), lambda i: (i, 0)),
        out_shape=jax.ShapeDtypeStruct((N_NODE, D_OUT), jnp.float32),
    )(wv, wv, den, x, w_out, b_out, sig)


def kernel(x_author, x_paper, edge_index_writes, edge_index_rev,
           W_kqv_author, b_kqv_author, W_kqv_paper, b_kqv_paper,
           W_k_rel, W_v_rel,
           W_out_author, b_out_author, W_out_paper, b_out_paper,
           skip_author, skip_paper, p_rel_writes, p_rel_rev):
    scale = 1.0 / math.sqrt(D)
    hidx = jnp.arange(H) * 2

    def fold(W_kqv, b_kqv, et, p_rel):
        Wk = W_kqv[:, 0:256].reshape(D_IN, H, D)
        Wq = W_kqv[:, 256:512].reshape(D_IN, H, D)
        Wv = W_kqv[:, 512:768].reshape(D_IN, H, D)
        bk = b_kqv[0:256].reshape(H, D)
        bq = b_kqv[256:512].reshape(H, D)
        bv = b_kqv[512:768].reshape(H, D)
        Rk = W_k_rel[hidx + et]
        Rv = W_v_rel[hidx + et]
        qs = (p_rel[0] * scale)[None, :, None]
        Wq2 = (Wq * qs).reshape(D_IN, 256)
        bq2 = (bq * qs[0]).reshape(256)
        Wk2 = jnp.einsum('ihd,hdo->iho', Wk, Rk).reshape(D_IN, 256)
        bk2 = jnp.einsum('hd,hdo->ho', bk, Rk).reshape(256)
        Wv2 = jnp.einsum('ihd,hdo->iho', Wv, Rv).reshape(D_IN, 256)
        bv2 = jnp.einsum('hd,hdo->ho', bv, Rv).reshape(256)
        W = jnp.concatenate([Wq2, Wk2, Wv2], axis=1)
        b = jnp.concatenate([bq2, bk2, bv2])[None, :]
        return W, b

    Wa, ba = fold(W_kqv_author, b_kqv_author, 0, p_rel_rev)
    Wp, bp = fold(W_kqv_paper, b_kqv_paper, 1, p_rel_writes)

    qa, ka, va0, va1 = _qkv(x_author, Wa, ba)
    qp, kp, vp0, vp1 = _qkv(x_paper, Wp, bp)

    q_cat = jnp.concatenate([qp, qa], axis=0)
    k_cat = jnp.concatenate([ka, kp], axis=0)
    v0_cat = jnp.concatenate([va0, vp0], axis=0)
    v1_cat = jnp.concatenate([va1, vp1], axis=0)
    src_cat = jnp.concatenate([edge_index_writes[0], edge_index_rev[0]])
    dst_cat = jnp.concatenate([edge_index_writes[1], edge_index_rev[1]])

    w_e, den = _edge_w(src_cat, dst_cat, q_cat, k_cat)
    wv = _edge_agg(src_cat, dst_cat, v0_cat, v1_cat, w_e)

    sig_a = jax.nn.sigmoid(skip_author)[0].reshape(1, 1)
    sig_p = jax.nn.sigmoid(skip_paper)[0].reshape(1, 1)
    out_a = _epilogue(wv, den, 1, x_author, W_out_author,
                      b_out_author[None, :], sig_a)
    out_p = _epilogue(wv, den, 0, x_paper, W_out_paper,
                      b_out_paper[None, :], sig_p)
    return out_a, out_p


# speedup vs baseline: 5.4739x; 1.2281x over previous
import functools
import math

import jax
import jax.numpy as jnp
from jax import lax
from jax.experimental import pallas as pl
from jax.experimental.pallas import tpu as pltpu
from jax.experimental.pallas import tpu_sc as plsc

N_NODE = 10000
E_PER = 160000
D_IN = 256
D_OUT = 256
H = 8
D = 32

NTILE = 16
TPT = E_PER // NTILE
CA = 80
CB = 80
N_PAD = 10240
RPT = N_PAD // NTILE


def _qkv_body(x_ref, w_ref, b_ref, q_ref, k_ref, v0_ref, v1_ref):
    acc = jnp.dot(x_ref[...], w_ref[...], preferred_element_type=jnp.float32)
    acc = acc + b_ref[...]
    q_ref[...] = acc[:, 0:256]
    k_ref[...] = acc[:, 256:512]
    v0_ref[...] = acc[:, 512:640]
    v1_ref[...] = acc[:, 640:768]


def _qkv(x, w, b):
    nb = 10
    rb = N_NODE // nb
    return pl.pallas_call(
        _qkv_body,
        grid=(nb,),
        in_specs=[
            pl.BlockSpec((rb, D_IN), lambda i: (i, 0)),
            pl.BlockSpec((D_IN, 3 * D_OUT), lambda i: (0, 0)),
            pl.BlockSpec((1, 3 * D_OUT), lambda i: (0, 0)),
        ],
        out_specs=[
            pl.BlockSpec((rb, D_OUT), lambda i: (i, 0)),
            pl.BlockSpec((rb, D_OUT), lambda i: (i, 0)),
            pl.BlockSpec((rb, 128), lambda i: (i, 0)),
            pl.BlockSpec((rb, 128), lambda i: (i, 0)),
        ],
        out_shape=[
            jax.ShapeDtypeStruct((N_NODE, D_OUT), jnp.float32),
            jax.ShapeDtypeStruct((N_NODE, D_OUT), jnp.float32),
            jax.ShapeDtypeStruct((N_NODE, 128), jnp.float32),
            jax.ShapeDtypeStruct((N_NODE, 128), jnp.float32),
        ],
    )(x, w, b)


@functools.lru_cache(maxsize=1)
def _sc_kernels():
  mesh = plsc.VectorSubcoreMesh(core_axis_name="c", subcore_axis_name="s")
  params = pltpu.CompilerParams(use_tc_tiling_on_sc=False,
                                needs_layout_passes=False)
  NCH = TPT // CA

  @functools.partial(
      pl.kernel,
      out_type=(
          jax.ShapeDtypeStruct((2 * E_PER, 16), jnp.float32),
          jax.ShapeDtypeStruct((2, N_PAD, 16), jnp.float32),
      ),
      mesh=mesh,
      compiler_params=params,
      scratch_types=[
          pltpu.VMEM((CA, 2), jnp.int32),
          pltpu.VMEM((CA, 2), jnp.int32),
          pltpu.VMEM((CA,), jnp.int32),
          pltpu.VMEM((CA,), jnp.int32),
          pltpu.VMEM((CA,), jnp.int32),
          pltpu.VMEM((CA,), jnp.int32),
          pltpu.VMEM((CA,), jnp.int32),
          pltpu.VMEM((CA,), jnp.int32),
          pltpu.VMEM((CA, 256), jnp.float32),
          pltpu.VMEM((CA, 256), jnp.float32),
          pltpu.VMEM((CA, 256), jnp.float32),
          pltpu.VMEM((CA, 256), jnp.float32),
          pltpu.VMEM((CA, 16), jnp.float32),
          pltpu.VMEM_SHARED((N_PAD, 16), jnp.float32),
          pltpu.SemaphoreType.DMA,
          pltpu.SemaphoreType.DMA,
          pltpu.SemaphoreType.DMA,
          pltpu.SemaphoreType.DMA,
      ],
  )
  def edge_w(rec_hbm, q_hbm, k_hbm, w_hbm, den_hbm,
             rec0, rec1, so0, so1, dr0, dr1, do0, do1,
             q0, q1, k0, k1, w_stage, den_sh,
             sq0, sq1, sk0, sk1):
    c = lax.axis_index("c")
    s = lax.axis_index("s")
    lane = lax.iota(jnp.int32, 16)
    z16 = jnp.zeros((16,), jnp.float32)
    col0 = jnp.zeros((16,), jnp.int32)
    col1 = jnp.full((16,), 1, jnp.int32)
    rec = (rec0, rec1)
    so = (so0, so1)
    dr = (dr0, dr1)
    do = (do0, do1)
    qb = (q0, q1)
    kb = (k0, k1)
    sq = (sq0, sq1)
    sk = (sk0, sk1)
    ebase = c * E_PER + s * TPT
    off = c * N_NODE
    rows_sl = pl.ds(s * RPT, RPT)

    def zw(j, carry):
      w_stage[j, :] = z16
      return carry
    lax.fori_loop(0, CA, zw, 0)
    for t in range(RPT // CA):
      pltpu.sync_copy(w_stage, den_sh.at[pl.ds(s * RPT + t * CA, CA)])
    plsc.subcore_barrier()

    def load_idx(g, b):
      base = ebase + g * CA
      pltpu.sync_copy(rec_hbm.at[pl.ds(base, CA)], rec[b])
      for t in range(CA // 16):
        rows = lane + t * 16
        sv_ = plsc.load_gather(rec[b], [rows, col0])
        dv_ = plsc.load_gather(rec[b], [rows, col1])
        sl = pl.ds(t * 16, 16)
        so[b][sl] = sv_ + off
        dr[b][sl] = dv_
        do[b][sl] = dv_ + off

    def gathers(b):
      pltpu.async_copy(q_hbm.at[do[b]], qb[b], sq[b])
      pltpu.async_copy(k_hbm.at[so[b]], kb[b], sk[b])

    def waitg(b):
      pltpu.make_async_copy(q_hbm.at[do[b]], qb[b], sq[b]).wait()
      pltpu.make_async_copy(k_hbm.at[so[b]], kb[b], sk[b]).wait()

    def compute(g, b):
      base = ebase + g * CA

      def grp(t, carry):
        rows = lane + t * 16
        for h in range(H):
          acc = z16
          for j in range(D):
            col = ((lane + j) & (D - 1)) + h * D
            qd = plsc.load_gather(qb[b], [rows, col])
            kd = plsc.load_gather(kb[b], [rows, col])
            acc = acc + qd * kd
          plsc.store_scatter(w_stage, [rows, jnp.full((16,), h, jnp.int32)],
                             jnp.exp(acc))
        return carry
      lax.fori_loop(0, CA // 16, grp, 0)
      pltpu.sync_copy(w_stage, w_hbm.at[pl.ds(base, CA)])
      pltpu.sync_copy(w_stage, den_sh.at[dr[b]], add=True)

    load_idx(jnp.int32(0), 0)
    gathers(0)

    def pair(t, carry):
      g = t * 2
      load_idx(g + 1, 1)
      gathers(1)
      waitg(0)
      compute(g, 0)
      load_idx(g + 2, 0)
      gathers(0)
      waitg(1)
      compute(g + 1, 1)
      return carry
    lax.fori_loop(0, (NCH - 1) // 2, pair, 0)
    waitg(0)
    compute(jnp.int32(NCH - 1), 0)

    plsc.subcore_barrier()
    pltpu.sync_copy(den_sh.at[rows_sl], den_hbm.at[c, rows_sl])

  @functools.partial(
      pl.kernel,
      out_type=jax.ShapeDtypeStruct((2, 2, N_PAD, 128), jnp.float32),
      mesh=mesh,
      compiler_params=params,
      scratch_types=[
          pltpu.VMEM((CA, 2), jnp.int32),
          pltpu.VMEM((CA, 2), jnp.int32),
          pltpu.VMEM((CA,), jnp.int32),
          pltpu.VMEM((CA,), jnp.int32),
          pltpu.VMEM((CA,), jnp.int32),
          pltpu.VMEM((CA,), jnp.int32),
          pltpu.VMEM((CA, 128), jnp.float32),
          pltpu.VMEM((CA, 128), jnp.float32),
          pltpu.VMEM((CA, 16), jnp.float32),
          pltpu.VMEM((CA, 16), jnp.float32),
          pltpu.VMEM((CA, 128), jnp.float32),
          pltpu.VMEM_SHARED((N_PAD, 128), jnp.float32),
          pltpu.SemaphoreType.DMA,
          pltpu.SemaphoreType.DMA,
          pltpu.SemaphoreType.DMA,
          pltpu.SemaphoreType.DMA,
      ],
  )
  def edge_agg(rec_hbm, v0_hbm, v1_hbm, w_hbm, out_hbm,
               rec0, rec1, so0, so1, dr0, dr1, v0, v1, wst0, wst1,
               wv_stage, acc_sh,
               sv0, sv1, sw0, sw1):
    c = lax.axis_index("c")
    s = lax.axis_index("s")
    lane = lax.iota(jnp.int32, 16)
    z16 = jnp.zeros((16,), jnp.float32)
    col0 = jnp.zeros((16,), jnp.int32)
    col1 = jnp.full((16,), 1, jnp.int32)
    rec = (rec0, rec1)
    so = (so0, so1)
    dr = (dr0, dr1)
    vb = (v0, v1)
    wb = (wst0, wst1)
    sv = (sv0, sv1)
    sw = (sw0, sw1)
    ebase = c * E_PER + s * TPT
    off = c * N_NODE
    rows_sl = pl.ds(s * RPT, RPT)

    def zwv(j, carry):
      for i in range(8):
        wv_stage[j, pl.ds(i * 16, 16)] = z16
      return carry

    def load_idx(g, b):
      base = ebase + g * CA
      pltpu.sync_copy(rec_hbm.at[pl.ds(base, CA)], rec[b])
      for t in range(CA // 16):
        rows = lane + t * 16
        sv_ = plsc.load_gather(rec[b], [rows, col0])
        dv_ = plsc.load_gather(rec[b], [rows, col1])
        sl = pl.ds(t * 16, 16)
        so[b][sl] = sv_ + off
        dr[b][sl] = dv_

    for half in range(2):
      lax.fori_loop(0, CA, zwv, 0)
      for t in range(RPT // CA):
        pltpu.sync_copy(wv_stage, acc_sh.at[pl.ds(s * RPT + t * CA, CA)])
      plsc.subcore_barrier()

      vtab = v0_hbm if half == 0 else v1_hbm

      def gathers(g, b):
        base = ebase + g * CA
        pltpu.async_copy(vtab.at[so[b]], vb[b], sv[b])
        pltpu.async_copy(w_hbm.at[pl.ds(base, CA)], wb[b], sw[b])

      def waitg(g, b):
        base = ebase + g * CA
        pltpu.make_async_copy(vtab.at[so[b]], vb[b], sv[b]).wait()
        pltpu.make_async_copy(w_hbm.at[pl.ds(base, CA)], wb[b], sw[b]).wait()

      def compute(g, b):
        def grp(t, carry):
          rows = lane + t * 16
          for hh in range(4):
            hcol = jnp.full((16,), half * 4 + hh, jnp.int32)
            wh = plsc.load_gather(wb[b], [rows, hcol])
            for j in range(D):
              col = ((lane + j) & (D - 1)) + hh * D
              vd = plsc.load_gather(vb[b], [rows, col])
              plsc.store_scatter(wv_stage, [rows, col], vd * wh)
          return carry
        lax.fori_loop(0, CA // 16, grp, 0)
        pltpu.sync_copy(wv_stage, acc_sh.at[dr[b]], add=True)

      load_idx(jnp.int32(0), 0)
      gathers(jnp.int32(0), 0)

      def pair(t, carry):
        g = t * 2
        load_idx(g + 1, 1)
        gathers(g + 1, 1)
        waitg(g, 0)
        compute(g, 0)
        load_idx(g + 2, 0)
        gathers(g + 2, 0)
        waitg(g + 1, 1)
        compute(g + 1, 1)
        return carry
      lax.fori_loop(0, (NCH - 1) // 2, pair, 0)
      waitg(jnp.int32(NCH - 1), 0)
      compute(jnp.int32(NCH - 1), 0)

      plsc.subcore_barrier()
      pltpu.sync_copy(acc_sh.at[rows_sl], out_hbm.at[c, half, rows_sl])
      plsc.subcore_barrier()

  return edge_w, edge_agg


def _edge_phase(rec_cat, q_cat, k_cat, v0_cat, v1_cat):
  ew, ea = _sc_kernels()
  w_e, den = ew(rec_cat, q_cat, k_cat)
  wv = ea(rec_cat, v0_cat, v1_cat, w_e)
  return w_e, den, wv


def _out_body(wv0_ref, wv1_ref, den_ref, x_ref, w_ref, b_ref, s_ref, o_ref):
    wv = jnp.concatenate([wv0_ref[0, 0], wv1_ref[0, 0]], axis=1)
    den = den_ref[0]
    parts = []
    for h in range(H):
        parts.append(wv[:, h * 32:(h + 1) * 32] / (den[:, h:h + 1] + 1e-16))
    agg = jnp.concatenate(parts, axis=1)
    g = 0.5 * agg * (1.0 + lax.erf(agg * (1.0 / math.sqrt(2.0))))
    y = jnp.dot(g, w_ref[...], preferred_element_type=jnp.float32) + b_ref[...]
    a = s_ref[0, 0]
    o_ref[...] = a * y + (1.0 - a) * x_ref[...]


def _epilogue(wv, den, etype, x, w_out, b_out, sig):
    nb = 10
    rb = N_NODE // nb
    return pl.pallas_call(
        _out_body,
        grid=(nb,),
        in_specs=[
            pl.BlockSpec((1, 1, rb, 128), lambda i: (etype, 0, i, 0)),
            pl.BlockSpec((1, 1, rb, 128), lambda i: (etype, 1, i, 0)),
            pl.BlockSpec((1, rb, 16), lambda i: (etype, i, 0)),
            pl.BlockSpec((rb, D_IN), lambda i: (i, 0)),
            pl.BlockSpec((D_OUT, D_OUT), lambda i: (0, 0)),
            pl.BlockSpec((1, D_OUT), lambda i: (0, 0)),
            pl.BlockSpec((1, 1), lambda i: (0, 0)),
        ],
        out_specs=pl.BlockSpec((rb, D_OUT), lambda i: (i, 0)),
        out_shape=jax.ShapeDtypeStruct((N_NODE, D_OUT), jnp.float32),
    )(wv, wv, den, x, w_out, b_out, sig)


def kernel(x_author, x_paper, edge_index_writes, edge_index_rev,
           W_kqv_author, b_kqv_author, W_kqv_paper, b_kqv_paper,
           W_k_rel, W_v_rel,
           W_out_author, b_out_author, W_out_paper, b_out_paper,
           skip_author, skip_paper, p_rel_writes, p_rel_rev):
    scale = 1.0 / math.sqrt(D)
    hidx = jnp.arange(H) * 2

    def fold(W_kqv, b_kqv, et, p_rel):
        Wk = W_kqv[:, 0:256].reshape(D_IN, H, D)
        Wq = W_kqv[:, 256:512].reshape(D_IN, H, D)
        Wv = W_kqv[:, 512:768].reshape(D_IN, H, D)
        bk = b_kqv[0:256].reshape(H, D)
        bq = b_kqv[256:512].reshape(H, D)
        bv = b_kqv[512:768].reshape(H, D)
        Rk = W_k_rel[hidx + et]
        Rv = W_v_rel[hidx + et]
        qs = (p_rel[0] * scale)[None, :, None]
        Wq2 = (Wq * qs).reshape(D_IN, 256)
        bq2 = (bq * qs[0]).reshape(256)
        Wk2 = jnp.einsum('ihd,hdo->iho', Wk, Rk).reshape(D_IN, 256)
        bk2 = jnp.einsum('hd,hdo->ho', bk, Rk).reshape(256)
        Wv2 = jnp.einsum('ihd,hdo->iho', Wv, Rv).reshape(D_IN, 256)
        bv2 = jnp.einsum('hd,hdo->ho', bv, Rv).reshape(256)
        W = jnp.concatenate([Wq2, Wk2, Wv2], axis=1)
        b = jnp.concatenate([bq2, bk2, bv2])[None, :]
        return W, b

    Wa, ba = fold(W_kqv_author, b_kqv_author, 0, p_rel_rev)
    Wp, bp = fold(W_kqv_paper, b_kqv_paper, 1, p_rel_writes)

    qa, ka, va0, va1 = _qkv(x_author, Wa, ba)
    qp, kp, vp0, vp1 = _qkv(x_paper, Wp, bp)

    q_cat = jnp.concatenate([qp, qa], axis=0)
    k_cat = jnp.concatenate([ka, kp], axis=0)
    v0_cat = jnp.concatenate([va0, vp0], axis=0)
    v1_cat = jnp.concatenate([va1, vp1], axis=0)
    rec_cat = jnp.concatenate(
        [edge_index_writes.T, edge_index_rev.T], axis=0)

    _, den, wv = _edge_phase(rec_cat, q_cat, k_cat, v0_cat, v1_cat)

    sig_a = jax.nn.sigmoid(skip_author)[0].reshape(1, 1)
    sig_p = jax.nn.sigmoid(skip_paper)[0].reshape(1, 1)
    out_a = _epilogue(wv, den, 1, x_author, W_out_author,
                      b_out_author[None, :], sig_a)
    out_p = _epilogue(wv, den, 0, x_paper, W_out_paper,
                      b_out_paper[None, :], sig_p)
    return out_a, out_p


# speedup vs baseline: 5.7970x; 1.0590x over previous
import functools
import math

import jax
import jax.numpy as jnp
from jax import lax
from jax.experimental import pallas as pl
from jax.experimental.pallas import tpu as pltpu
from jax.experimental.pallas import tpu_sc as plsc

N_NODE = 10000
E_PER = 160000
D_IN = 256
D_OUT = 256
H = 8
D = 32

NTILE = 16
TPT = E_PER // NTILE
CA = 80
CB = 80
N_PAD = 10240
RPT = N_PAD // NTILE


def _qkv_body(x_ref, w_ref, b_ref, q_ref, k_ref, v0_ref, v1_ref):
    acc = jnp.dot(x_ref[...], w_ref[...], preferred_element_type=jnp.float32)
    acc = acc + b_ref[...]
    q_ref[...] = acc[:, 0:256]
    k_ref[...] = acc[:, 256:512]
    v0_ref[...] = acc[:, 512:640]
    v1_ref[...] = acc[:, 640:768]


def _qkv(x, w, b):
    nb = 10
    rb = N_NODE // nb
    return pl.pallas_call(
        _qkv_body,
        grid=(nb,),
        in_specs=[
            pl.BlockSpec((rb, D_IN), lambda i: (i, 0)),
            pl.BlockSpec((D_IN, 3 * D_OUT), lambda i: (0, 0)),
            pl.BlockSpec((1, 3 * D_OUT), lambda i: (0, 0)),
        ],
        out_specs=[
            pl.BlockSpec((rb, D_OUT), lambda i: (i, 0)),
            pl.BlockSpec((rb, D_OUT), lambda i: (i, 0)),
            pl.BlockSpec((rb, 128), lambda i: (i, 0)),
            pl.BlockSpec((rb, 128), lambda i: (i, 0)),
        ],
        out_shape=[
            jax.ShapeDtypeStruct((N_NODE, D_OUT), jnp.float32),
            jax.ShapeDtypeStruct((N_NODE, D_OUT), jnp.float32),
            jax.ShapeDtypeStruct((N_NODE, 128), jnp.float32),
            jax.ShapeDtypeStruct((N_NODE, 128), jnp.float32),
        ],
    )(x, w, b)


@functools.lru_cache(maxsize=1)
def _sc_kernels():
  mesh = plsc.VectorSubcoreMesh(core_axis_name="c", subcore_axis_name="s")
  params = pltpu.CompilerParams(use_tc_tiling_on_sc=False,
                                needs_layout_passes=False)
  NCH = TPT // CA

  @functools.partial(
      pl.kernel,
      out_type=(
          jax.ShapeDtypeStruct((2 * E_PER, 16), jnp.float32),
          jax.ShapeDtypeStruct((2, N_PAD, 16), jnp.float32),
      ),
      mesh=mesh,
      compiler_params=params,
      scratch_types=[
          pltpu.VMEM((CA, 2), jnp.int32),
          pltpu.VMEM((CA, 2), jnp.int32),
          pltpu.VMEM((CA,), jnp.int32),
          pltpu.VMEM((CA,), jnp.int32),
          pltpu.VMEM((CA,), jnp.int32),
          pltpu.VMEM((CA,), jnp.int32),
          pltpu.VMEM((CA,), jnp.int32),
          pltpu.VMEM((CA,), jnp.int32),
          pltpu.VMEM((CA, 256), jnp.float32),
          pltpu.VMEM((CA, 256), jnp.float32),
          pltpu.VMEM((CA, 256), jnp.float32),
          pltpu.VMEM((CA, 256), jnp.float32),
          pltpu.VMEM((CA, 16), jnp.float32),
          pltpu.VMEM((CA, 16), jnp.float32),
          pltpu.VMEM((CA,), jnp.int32),
          pltpu.VMEM((CA,), jnp.int32),
          pltpu.VMEM_SHARED((N_PAD, 16), jnp.float32),
          pltpu.SemaphoreType.DMA,
          pltpu.SemaphoreType.DMA,
          pltpu.SemaphoreType.DMA,
          pltpu.SemaphoreType.DMA,
          pltpu.SemaphoreType.DMA,
          pltpu.SemaphoreType.DMA,
          pltpu.SemaphoreType.DMA,
          pltpu.SemaphoreType.DMA,
      ],
  )
  def edge_w(rec_hbm, q_hbm, k_hbm, w_hbm, den_hbm,
             rec0, rec1, so0, so1, dr0, dr1, do0, do1,
             q0, q1, k0, k1, wst0, wst1, ds0, ds1, den_sh,
             sq0, sq1, sk0, sk1, swr0, swr1, sds0, sds1):
    c = lax.axis_index("c")
    s = lax.axis_index("s")
    lane = lax.iota(jnp.int32, 16)
    z16 = jnp.zeros((16,), jnp.float32)
    col0 = jnp.zeros((16,), jnp.int32)
    col1 = jnp.full((16,), 1, jnp.int32)
    rec = (rec0, rec1)
    so = (so0, so1)
    dr = (dr0, dr1)
    do = (do0, do1)
    qb = (q0, q1)
    kb = (k0, k1)
    wsb = (wst0, wst1)
    dsb = (ds0, ds1)
    sq = (sq0, sq1)
    sk = (sk0, sk1)
    swr = (swr0, swr1)
    sds = (sds0, sds1)
    ebase = c * E_PER + s * TPT
    off = c * N_NODE
    rows_sl = pl.ds(s * RPT, RPT)

    def zw(j, carry):
      wst0[j, :] = z16
      return carry
    lax.fori_loop(0, CA, zw, 0)
    for t in range(RPT // CA):
      pltpu.sync_copy(wst0, den_sh.at[pl.ds(s * RPT + t * CA, CA)])
    plsc.subcore_barrier()

    def load_idx(g, b):
      base = ebase + g * CA
      pltpu.sync_copy(rec_hbm.at[pl.ds(base, CA)], rec[b])
      for t in range(CA // 16):
        rows = lane + t * 16
        sv_ = plsc.load_gather(rec[b], [rows, col0])
        dv_ = plsc.load_gather(rec[b], [rows, col1])
        sl = pl.ds(t * 16, 16)
        so[b][sl] = sv_ + off
        dr[b][sl] = dv_
        do[b][sl] = dv_ + off

    def gathers(b):
      pltpu.async_copy(q_hbm.at[do[b]], qb[b], sq[b])
      pltpu.async_copy(k_hbm.at[so[b]], kb[b], sk[b])

    def waitg(b):
      pltpu.make_async_copy(q_hbm.at[do[b]], qb[b], sq[b]).wait()
      pltpu.make_async_copy(k_hbm.at[so[b]], kb[b], sk[b]).wait()

    def drain_stage(g, b):
      base = ebase + (g - 2) * CA
      pltpu.make_async_copy(wsb[b], w_hbm.at[pl.ds(base, CA)],
                            swr[b]).wait()
      pltpu.make_async_copy(wsb[b], den_sh.at[dsb[b]], sds[b]).wait()

    def compute(g, b, first):
      base = ebase + g * CA
      if not first:
        @pl.when(g >= 2)
        def _():
          drain_stage(g, b)

      def grp(t, carry):
        rows = lane + t * 16
        for h in range(H):
          acc = z16
          for j in range(D):
            col = ((lane + j) & (D - 1)) + h * D
            qd = plsc.load_gather(qb[b], [rows, col])
            kd = plsc.load_gather(kb[b], [rows, col])
            acc = acc + qd * kd
          plsc.store_scatter(wsb[b], [rows, jnp.full((16,), h, jnp.int32)],
                             jnp.exp(acc))
        return carry
      lax.fori_loop(0, CA // 16, grp, 0)
      for t in range(CA // 16):
        sl = pl.ds(t * 16, 16)
        dsb[b][sl] = dr[b][sl]
      pltpu.async_copy(wsb[b], w_hbm.at[pl.ds(base, CA)], swr[b])
      pltpu.async_copy(wsb[b], den_sh.at[dsb[b]], sds[b], add=True)

    load_idx(jnp.int32(0), 0)
    gathers(0)
    compute0_done = False

    def pair(t, carry):
      g = t * 2
      load_idx(g + 1, 1)
      gathers(1)
      waitg(0)
      compute(g, 0, False)
      load_idx(g + 2, 0)
      gathers(0)
      waitg(1)
      compute(g + 1, 1, False)
      return carry
    lax.fori_loop(0, (NCH - 1) // 2, pair, 0)
    waitg(0)
    compute(jnp.int32(NCH - 1), 0, False)
    drain_stage(jnp.int32(NCH), 1)
    drain_stage(jnp.int32(NCH + 1), 0)

    plsc.subcore_barrier()
    pltpu.sync_copy(den_sh.at[rows_sl], den_hbm.at[c, rows_sl])

  @functools.partial(
      pl.kernel,
      out_type=jax.ShapeDtypeStruct((2, 2, N_PAD, 128), jnp.float32),
      mesh=mesh,
      compiler_params=params,
      scratch_types=[
          pltpu.VMEM((CA, 2), jnp.int32),
          pltpu.VMEM((CA, 2), jnp.int32),
          pltpu.VMEM((CA,), jnp.int32),
          pltpu.VMEM((CA,), jnp.int32),
          pltpu.VMEM((CA,), jnp.int32),
          pltpu.VMEM((CA,), jnp.int32),
          pltpu.VMEM((CA, 128), jnp.float32),
          pltpu.VMEM((CA, 128), jnp.float32),
          pltpu.VMEM((CA, 16), jnp.float32),
          pltpu.VMEM((CA, 16), jnp.float32),
          pltpu.VMEM((CA, 128), jnp.float32),
          pltpu.VMEM((CA, 128), jnp.float32),
          pltpu.VMEM((CA,), jnp.int32),
          pltpu.VMEM((CA,), jnp.int32),
          pltpu.VMEM_SHARED((N_PAD, 128), jnp.float32),
          pltpu.SemaphoreType.DMA,
          pltpu.SemaphoreType.DMA,
          pltpu.SemaphoreType.DMA,
          pltpu.SemaphoreType.DMA,
          pltpu.SemaphoreType.DMA,
          pltpu.SemaphoreType.DMA,
      ],
  )
  def edge_agg(rec_hbm, v0_hbm, v1_hbm, w_hbm, out_hbm,
               rec0, rec1, so0, so1, dr0, dr1, v0, v1, wst0, wst1,
               wv0, wv1, ds0, ds1, acc_sh,
               sv0, sv1, sw0, sw1, sc0, sc1):
    c = lax.axis_index("c")
    s = lax.axis_index("s")
    lane = lax.iota(jnp.int32, 16)
    z16 = jnp.zeros((16,), jnp.float32)
    col0 = jnp.zeros((16,), jnp.int32)
    col1 = jnp.full((16,), 1, jnp.int32)
    rec = (rec0, rec1)
    so = (so0, so1)
    dr = (dr0, dr1)
    vb = (v0, v1)
    wb = (wst0, wst1)
    wvb = (wv0, wv1)
    dsb = (ds0, ds1)
    sv = (sv0, sv1)
    sw = (sw0, sw1)
    scs = (sc0, sc1)
    ebase = c * E_PER + s * TPT
    off = c * N_NODE
    rows_sl = pl.ds(s * RPT, RPT)

    def zwv(j, carry):
      for i in range(8):
        wv0[j, pl.ds(i * 16, 16)] = z16
      return carry

    def load_idx(g, b):
      base = ebase + g * CA
      pltpu.sync_copy(rec_hbm.at[pl.ds(base, CA)], rec[b])
      for t in range(CA // 16):
        rows = lane + t * 16
        sv_ = plsc.load_gather(rec[b], [rows, col0])
        dv_ = plsc.load_gather(rec[b], [rows, col1])
        sl = pl.ds(t * 16, 16)
        so[b][sl] = sv_ + off
        dr[b][sl] = dv_

    for half in range(2):
      lax.fori_loop(0, CA, zwv, 0)
      for t in range(RPT // CA):
        pltpu.sync_copy(wv0, acc_sh.at[pl.ds(s * RPT + t * CA, CA)])
      plsc.subcore_barrier()

      vtab = v0_hbm if half == 0 else v1_hbm

      def gathers(g, b):
        base = ebase + g * CA
        pltpu.async_copy(vtab.at[so[b]], vb[b], sv[b])
        pltpu.async_copy(w_hbm.at[pl.ds(base, CA)], wb[b], sw[b])

      def waitg(g, b):
        base = ebase + g * CA
        pltpu.make_async_copy(vtab.at[so[b]], vb[b], sv[b]).wait()
        pltpu.make_async_copy(w_hbm.at[pl.ds(base, CA)], wb[b], sw[b]).wait()

      def drain_sc(b):
        pltpu.make_async_copy(wvb[b], acc_sh.at[dsb[b]], scs[b]).wait()

      def compute(g, b):
        @pl.when(g >= 2)
        def _():
          drain_sc(b)

        def grp(t, carry):
          rows = lane + t * 16
          for hh in range(4):
            hcol = jnp.full((16,), half * 4 + hh, jnp.int32)
            wh = plsc.load_gather(wb[b], [rows, hcol])
            for j in range(D):
              col = ((lane + j) & (D - 1)) + hh * D
              vd = plsc.load_gather(vb[b], [rows, col])
              plsc.store_scatter(wvb[b], [rows, col], vd * wh)
          return carry
        lax.fori_loop(0, CA // 16, grp, 0)
        for t in range(CA // 16):
          sl = pl.ds(t * 16, 16)
          dsb[b][sl] = dr[b][sl]
        pltpu.async_copy(wvb[b], acc_sh.at[dsb[b]], scs[b], add=True)

      load_idx(jnp.int32(0), 0)
      gathers(jnp.int32(0), 0)

      def pair(t, carry):
        g = t * 2
        load_idx(g + 1, 1)
        gathers(g + 1, 1)
        waitg(g, 0)
        compute(g, 0)
        load_idx(g + 2, 0)
        gathers(g + 2, 0)
        waitg(g + 1, 1)
        compute(g + 1, 1)
        return carry
      lax.fori_loop(0, (NCH - 1) // 2, pair, 0)
      waitg(jnp.int32(NCH - 1), 0)
      compute(jnp.int32(NCH - 1), 0)
      drain_sc(1)
      drain_sc(0)

      plsc.subcore_barrier()
      pltpu.sync_copy(acc_sh.at[rows_sl], out_hbm.at[c, half, rows_sl])
      plsc.subcore_barrier()

  return edge_w, edge_agg


def _edge_phase(rec_cat, q_cat, k_cat, v0_cat, v1_cat):
  ew, ea = _sc_kernels()
  w_e, den = ew(rec_cat, q_cat, k_cat)
  wv = ea(rec_cat, v0_cat, v1_cat, w_e)
  return w_e, den, wv


def _out_body(wv0_ref, wv1_ref, den_ref, x_ref, w_ref, b_ref, s_ref, o_ref):
    wv = jnp.concatenate([wv0_ref[0, 0], wv1_ref[0, 0]], axis=1)
    den = den_ref[0]
    parts = []
    for h in range(H):
        parts.append(wv[:, h * 32:(h + 1) * 32] / (den[:, h:h + 1] + 1e-16))
    agg = jnp.concatenate(parts, axis=1)
    g = 0.5 * agg * (1.0 + lax.erf(agg * (1.0 / math.sqrt(2.0))))
    y = jnp.dot(g, w_ref[...], preferred_element_type=jnp.float32) + b_ref[...]
    a = s_ref[0, 0]
    o_ref[...] = a * y + (1.0 - a) * x_ref[...]


def _epilogue(wv, den, etype, x, w_out, b_out, sig):
    nb = 10
    rb = N_NODE // nb
    return pl.pallas_call(
        _out_body,
        grid=(nb,),
        in_specs=[
            pl.BlockSpec((1, 1, rb, 128), lambda i: (etype, 0, i, 0)),
            pl.BlockSpec((1, 1, rb, 128), lambda i: (etype, 1, i, 0)),
            pl.BlockSpec((1, rb, 16), lambda i: (etype, i, 0)),
            pl.BlockSpec((rb, D_IN), lambda i: (i, 0)),
            pl.BlockSpec((D_OUT, D_OUT), lambda i: (0, 0)),
            pl.BlockSpec((1, D_OUT), lambda i: (0, 0)),
            pl.BlockSpec((1, 1), lambda i: (0, 0)),
        ],
        out_specs=pl.BlockSpec((rb, D_OUT), lambda i: (i, 0)),
        out_shape=jax.ShapeDtypeStruct((N_NODE, D_OUT), jnp.float32),
    )(wv, wv, den, x, w_out, b_out, sig)


def kernel(x_author, x_paper, edge_index_writes, edge_index_rev,
           W_kqv_author, b_kqv_author, W_kqv_paper, b_kqv_paper,
           W_k_rel, W_v_rel,
           W_out_author, b_out_author, W_out_paper, b_out_paper,
           skip_author, skip_paper, p_rel_writes, p_rel_rev):
    scale = 1.0 / math.sqrt(D)
    hidx = jnp.arange(H) * 2

    def fold(W_kqv, b_kqv, et, p_rel):
        Wk = W_kqv[:, 0:256].reshape(D_IN, H, D)
        Wq = W_kqv[:, 256:512].reshape(D_IN, H, D)
        Wv = W_kqv[:, 512:768].reshape(D_IN, H, D)
        bk = b_kqv[0:256].reshape(H, D)
        bq = b_kqv[256:512].reshape(H, D)
        bv = b_kqv[512:768].reshape(H, D)
        Rk = W_k_rel[hidx + et]
        Rv = W_v_rel[hidx + et]
        qs = (p_rel[0] * scale)[None, :, None]
        Wq2 = (Wq * qs).reshape(D_IN, 256)
        bq2 = (bq * qs[0]).reshape(256)
        Wk2 = jnp.einsum('ihd,hdo->iho', Wk, Rk).reshape(D_IN, 256)
        bk2 = jnp.einsum('hd,hdo->ho', bk, Rk).reshape(256)
        Wv2 = jnp.einsum('ihd,hdo->iho', Wv, Rv).reshape(D_IN, 256)
        bv2 = jnp.einsum('hd,hdo->ho', bv, Rv).reshape(256)
        W = jnp.concatenate([Wq2, Wk2, Wv2], axis=1)
        b = jnp.concatenate([bq2, bk2, bv2])[None, :]
        return W, b

    Wa, ba = fold(W_kqv_author, b_kqv_author, 0, p_rel_rev)
    Wp, bp = fold(W_kqv_paper, b_kqv_paper, 1, p_rel_writes)

    qa, ka, va0, va1 = _qkv(x_author, Wa, ba)
    qp, kp, vp0, vp1 = _qkv(x_paper, Wp, bp)

    q_cat = jnp.concatenate([qp, qa], axis=0)
    k_cat = jnp.concatenate([ka, kp], axis=0)
    v0_cat = jnp.concatenate([va0, vp0], axis=0)
    v1_cat = jnp.concatenate([va1, vp1], axis=0)
    rec_cat = jnp.concatenate(
        [edge_index_writes.T, edge_index_rev.T], axis=0)

    _, den, wv = _edge_phase(rec_cat, q_cat, k_cat, v0_cat, v1_cat)

    sig_a = jax.nn.sigmoid(skip_author)[0].reshape(1, 1)
    sig_p = jax.nn.sigmoid(skip_paper)[0].reshape(1, 1)
    out_a = _epilogue(wv, den, 1, x_author, W_out_author,
                      b_out_author[None, :], sig_a)
    out_p = _epilogue(wv, den, 0, x_paper, W_out_paper,
                      b_out_paper[None, :], sig_p)
    return out_a, out_p


# speedup vs baseline: 5.9057x; 1.0188x over previous
import functools
import math

import jax
import jax.numpy as jnp
from jax import lax
from jax.experimental import pallas as pl
from jax.experimental.pallas import tpu as pltpu
from jax.experimental.pallas import tpu_sc as plsc

N_NODE = 10000
E_PER = 160000
D_IN = 256
D_OUT = 256
H = 8
D = 32

NTILE = 16
TPT = E_PER // NTILE
CA = 80
CB = 80
N_PAD = 10240
RPT = N_PAD // NTILE


def _qkv_body(xp_ref, xa_ref, wq_ref, bq_ref, wkv_ref, bkv_ref,
              q_ref, k_ref, v0_ref, v1_ref):
    t = pl.program_id(0)
    x_dst = jnp.where(t == 0, xp_ref[...], xa_ref[...])
    x_src = jnp.where(t == 0, xa_ref[...], xp_ref[...])
    q = jnp.dot(x_dst, wq_ref[0], preferred_element_type=jnp.float32)
    q_ref[0] = q + bq_ref[0]
    kv = jnp.dot(x_src, wkv_ref[0], preferred_element_type=jnp.float32)
    kv = kv + bkv_ref[0]
    k_ref[0] = kv[:, 0:256]
    v0_ref[0] = kv[:, 256:384]
    v1_ref[0] = kv[:, 384:512]


def _qkv(x_author, x_paper, wq, bq, wkv, bkv):
    nb = 10
    rb = N_NODE // nb
    outs = pl.pallas_call(
        _qkv_body,
        grid=(2, nb),
        in_specs=[
            pl.BlockSpec((rb, D_IN), lambda t, i: (i, 0)),
            pl.BlockSpec((rb, D_IN), lambda t, i: (i, 0)),
            pl.BlockSpec((1, D_IN, 256), lambda t, i: (t, 0, 0)),
            pl.BlockSpec((1, 1, 256), lambda t, i: (t, 0, 0)),
            pl.BlockSpec((1, D_IN, 512), lambda t, i: (t, 0, 0)),
            pl.BlockSpec((1, 1, 512), lambda t, i: (t, 0, 0)),
        ],
        out_specs=[
            pl.BlockSpec((1, rb, 256), lambda t, i: (t, i, 0)),
            pl.BlockSpec((1, rb, 256), lambda t, i: (t, i, 0)),
            pl.BlockSpec((1, rb, 128), lambda t, i: (t, i, 0)),
            pl.BlockSpec((1, rb, 128), lambda t, i: (t, i, 0)),
        ],
        out_shape=[
            jax.ShapeDtypeStruct((2, N_NODE, 256), jnp.float32),
            jax.ShapeDtypeStruct((2, N_NODE, 256), jnp.float32),
            jax.ShapeDtypeStruct((2, N_NODE, 128), jnp.float32),
            jax.ShapeDtypeStruct((2, N_NODE, 128), jnp.float32),
        ],
    )(x_paper, x_author, wq, bq, wkv, bkv)
    q, k, v0, v1 = outs
    return (q.reshape(2 * N_NODE, 256), k.reshape(2 * N_NODE, 256),
            v0.reshape(2 * N_NODE, 128), v1.reshape(2 * N_NODE, 128))


@functools.lru_cache(maxsize=1)
def _sc_kernels():
  mesh = plsc.VectorSubcoreMesh(core_axis_name="c", subcore_axis_name="s")
  params = pltpu.CompilerParams(use_tc_tiling_on_sc=False,
                                needs_layout_passes=False)
  NCH = TPT // CA

  @functools.partial(
      pl.kernel,
      out_type=(
          jax.ShapeDtypeStruct((2 * E_PER, 16), jnp.float32),
          jax.ShapeDtypeStruct((2, N_PAD, 16), jnp.float32),
      ),
      mesh=mesh,
      compiler_params=params,
      scratch_types=[
          pltpu.VMEM((CA, 2), jnp.int32),
          pltpu.VMEM((CA, 2), jnp.int32),
          pltpu.VMEM((CA,), jnp.int32),
          pltpu.VMEM((CA,), jnp.int32),
          pltpu.VMEM((CA,), jnp.int32),
          pltpu.VMEM((CA,), jnp.int32),
          pltpu.VMEM((CA,), jnp.int32),
          pltpu.VMEM((CA,), jnp.int32),
          pltpu.VMEM((CA, 256), jnp.float32),
          pltpu.VMEM((CA, 256), jnp.float32),
          pltpu.VMEM((CA, 256), jnp.float32),
          pltpu.VMEM((CA, 256), jnp.float32),
          pltpu.VMEM((CA, 16), jnp.float32),
          pltpu.VMEM((CA, 16), jnp.float32),
          pltpu.VMEM((CA,), jnp.int32),
          pltpu.VMEM((CA,), jnp.int32),
          pltpu.VMEM_SHARED((N_PAD, 16), jnp.float32),
          pltpu.SemaphoreType.DMA,
          pltpu.SemaphoreType.DMA,
          pltpu.SemaphoreType.DMA,
          pltpu.SemaphoreType.DMA,
          pltpu.SemaphoreType.DMA,
          pltpu.SemaphoreType.DMA,
          pltpu.SemaphoreType.DMA,
          pltpu.SemaphoreType.DMA,
      ],
  )
  def edge_w(rec_hbm, q_hbm, k_hbm, w_hbm, den_hbm,
             rec0, rec1, so0, so1, dr0, dr1, do0, do1,
             q0, q1, k0, k1, wst0, wst1, ds0, ds1, den_sh,
             sq0, sq1, sk0, sk1, swr0, swr1, sds0, sds1):
    c = lax.axis_index("c")
    s = lax.axis_index("s")
    lane = lax.iota(jnp.int32, 16)
    z16 = jnp.zeros((16,), jnp.float32)
    col0 = jnp.zeros((16,), jnp.int32)
    col1 = jnp.full((16,), 1, jnp.int32)
    rec = (rec0, rec1)
    so = (so0, so1)
    dr = (dr0, dr1)
    do = (do0, do1)
    qb = (q0, q1)
    kb = (k0, k1)
    wsb = (wst0, wst1)
    dsb = (ds0, ds1)
    sq = (sq0, sq1)
    sk = (sk0, sk1)
    swr = (swr0, swr1)
    sds = (sds0, sds1)
    ebase = c * E_PER + s * TPT
    off = c * N_NODE
    rows_sl = pl.ds(s * RPT, RPT)

    def zw(j, carry):
      wst0[j, :] = z16
      return carry
    lax.fori_loop(0, CA, zw, 0)
    for t in range(RPT // CA):
      pltpu.sync_copy(wst0, den_sh.at[pl.ds(s * RPT + t * CA, CA)])
    plsc.subcore_barrier()

    def load_idx(g, b):
      base = ebase + g * CA
      pltpu.sync_copy(rec_hbm.at[pl.ds(base, CA)], rec[b])
      for t in range(CA // 16):
        rows = lane + t * 16
        sv_ = plsc.load_gather(rec[b], [rows, col0])
        dv_ = plsc.load_gather(rec[b], [rows, col1])
        sl = pl.ds(t * 16, 16)
        so[b][sl] = sv_ + off
        dr[b][sl] = dv_
        do[b][sl] = dv_ + off

    def gathers(b):
      pltpu.async_copy(q_hbm.at[do[b]], qb[b], sq[b])
      pltpu.async_copy(k_hbm.at[so[b]], kb[b], sk[b])

    def waitg(b):
      pltpu.make_async_copy(q_hbm.at[do[b]], qb[b], sq[b]).wait()
      pltpu.make_async_copy(k_hbm.at[so[b]], kb[b], sk[b]).wait()

    def drain_stage(g, b):
      base = ebase + (g - 2) * CA
      pltpu.make_async_copy(wsb[b], w_hbm.at[pl.ds(base, CA)],
                            swr[b]).wait()
      pltpu.make_async_copy(wsb[b], den_sh.at[dsb[b]], sds[b]).wait()

    def compute(g, b, first):
      base = ebase + g * CA
      if not first:
        @pl.when(g >= 2)
        def _():
          drain_stage(g, b)

      def grp(t, carry):
        rows = lane + t * 16
        for h in range(H):
          acc = z16
          for j in range(D):
            col = ((lane + j) & (D - 1)) + h * D
            qd = plsc.load_gather(qb[b], [rows, col])
            kd = plsc.load_gather(kb[b], [rows, col])
            acc = acc + qd * kd
          plsc.store_scatter(wsb[b], [rows, jnp.full((16,), h, jnp.int32)],
                             jnp.exp(acc))
        return carry
      lax.fori_loop(0, CA // 16, grp, 0)
      for t in range(CA // 16):
        sl = pl.ds(t * 16, 16)
        dsb[b][sl] = dr[b][sl]
      pltpu.async_copy(wsb[b], w_hbm.at[pl.ds(base, CA)], swr[b])
      pltpu.async_copy(wsb[b], den_sh.at[dsb[b]], sds[b], add=True)

    load_idx(jnp.int32(0), 0)
    gathers(0)
    compute0_done = False

    def pair(t, carry):
      g = t * 2
      load_idx(g + 1, 1)
      gathers(1)
      waitg(0)
      compute(g, 0, False)
      load_idx(g + 2, 0)
      gathers(0)
      waitg(1)
      compute(g + 1, 1, False)
      return carry
    lax.fori_loop(0, (NCH - 1) // 2, pair, 0)
    waitg(0)
    compute(jnp.int32(NCH - 1), 0, False)
    drain_stage(jnp.int32(NCH), 1)
    drain_stage(jnp.int32(NCH + 1), 0)

    plsc.subcore_barrier()
    pltpu.sync_copy(den_sh.at[rows_sl], den_hbm.at[c, rows_sl])

  @functools.partial(
      pl.kernel,
      out_type=jax.ShapeDtypeStruct((2, 2, N_PAD, 128), jnp.float32),
      mesh=mesh,
      compiler_params=params,
      scratch_types=[
          pltpu.VMEM((CA, 2), jnp.int32),
          pltpu.VMEM((CA, 2), jnp.int32),
          pltpu.VMEM((CA,), jnp.int32),
          pltpu.VMEM((CA,), jnp.int32),
          pltpu.VMEM((CA,), jnp.int32),
          pltpu.VMEM((CA,), jnp.int32),
          pltpu.VMEM((CA, 128), jnp.float32),
          pltpu.VMEM((CA, 128), jnp.float32),
          pltpu.VMEM((CA, 16), jnp.float32),
          pltpu.VMEM((CA, 16), jnp.float32),
          pltpu.VMEM((CA, 128), jnp.float32),
          pltpu.VMEM((CA, 128), jnp.float32),
          pltpu.VMEM((CA,), jnp.int32),
          pltpu.VMEM((CA,), jnp.int32),
          pltpu.VMEM_SHARED((N_PAD, 128), jnp.float32),
          pltpu.SemaphoreType.DMA,
          pltpu.SemaphoreType.DMA,
          pltpu.SemaphoreType.DMA,
          pltpu.SemaphoreType.DMA,
          pltpu.SemaphoreType.DMA,
          pltpu.SemaphoreType.DMA,
      ],
  )
  def edge_agg(rec_hbm, v0_hbm, v1_hbm, w_hbm, out_hbm,
               rec0, rec1, so0, so1, dr0, dr1, v0, v1, wst0, wst1,
               wv0, wv1, ds0, ds1, acc_sh,
               sv0, sv1, sw0, sw1, sc0, sc1):
    c = lax.axis_index("c")
    s = lax.axis_index("s")
    lane = lax.iota(jnp.int32, 16)
    z16 = jnp.zeros((16,), jnp.float32)
    col0 = jnp.zeros((16,), jnp.int32)
    col1 = jnp.full((16,), 1, jnp.int32)
    rec = (rec0, rec1)
    so = (so0, so1)
    dr = (dr0, dr1)
    vb = (v0, v1)
    wb = (wst0, wst1)
    wvb = (wv0, wv1)
    dsb = (ds0, ds1)
    sv = (sv0, sv1)
    sw = (sw0, sw1)
    scs = (sc0, sc1)
    ebase = c * E_PER + s * TPT
    off = c * N_NODE
    rows_sl = pl.ds(s * RPT, RPT)

    def zwv(j, carry):
      for i in range(8):
        wv0[j, pl.ds(i * 16, 16)] = z16
      return carry

    def load_idx(g, b):
      base = ebase + g * CA
      pltpu.sync_copy(rec_hbm.at[pl.ds(base, CA)], rec[b])
      for t in range(CA // 16):
        rows = lane + t * 16
        sv_ = plsc.load_gather(rec[b], [rows, col0])
        dv_ = plsc.load_gather(rec[b], [rows, col1])
        sl = pl.ds(t * 16, 16)
        so[b][sl] = sv_ + off
        dr[b][sl] = dv_

    for half in range(2):
      lax.fori_loop(0, CA, zwv, 0)
      for t in range(RPT // CA):
        pltpu.sync_copy(wv0, acc_sh.at[pl.ds(s * RPT + t * CA, CA)])
      plsc.subcore_barrier()

      vtab = v0_hbm if half == 0 else v1_hbm

      def gathers(g, b):
        base = ebase + g * CA
        pltpu.async_copy(vtab.at[so[b]], vb[b], sv[b])
        pltpu.async_copy(w_hbm.at[pl.ds(base, CA)], wb[b], sw[b])

      def waitg(g, b):
        base = ebase + g * CA
        pltpu.make_async_copy(vtab.at[so[b]], vb[b], sv[b]).wait()
        pltpu.make_async_copy(w_hbm.at[pl.ds(base, CA)], wb[b], sw[b]).wait()

      def drain_sc(b):
        pltpu.make_async_copy(wvb[b], acc_sh.at[dsb[b]], scs[b]).wait()

      def compute(g, b):
        @pl.when(g >= 2)
        def _():
          drain_sc(b)

        def grp(t, carry):
          rows = lane + t * 16
          for hh in range(4):
            hcol = jnp.full((16,), half * 4 + hh, jnp.int32)
            wh = plsc.load_gather(wb[b], [rows, hcol])
            for j in range(D):
              col = ((lane + j) & (D - 1)) + hh * D
              vd = plsc.load_gather(vb[b], [rows, col])
              plsc.store_scatter(wvb[b], [rows, col], vd * wh)
          return carry
        lax.fori_loop(0, CA // 16, grp, 0)
        for t in range(CA // 16):
          sl = pl.ds(t * 16, 16)
          dsb[b][sl] = dr[b][sl]
        pltpu.async_copy(wvb[b], acc_sh.at[dsb[b]], scs[b], add=True)

      load_idx(jnp.int32(0), 0)
      gathers(jnp.int32(0), 0)

      def pair(t, carry):
        g = t * 2
        load_idx(g + 1, 1)
        gathers(g + 1, 1)
        waitg(g, 0)
        compute(g, 0)
        load_idx(g + 2, 0)
        gathers(g + 2, 0)
        waitg(g + 1, 1)
        compute(g + 1, 1)
        return carry
      lax.fori_loop(0, (NCH - 1) // 2, pair, 0)
      waitg(jnp.int32(NCH - 1), 0)
      compute(jnp.int32(NCH - 1), 0)
      drain_sc(1)
      drain_sc(0)

      plsc.subcore_barrier()
      pltpu.sync_copy(acc_sh.at[rows_sl], out_hbm.at[c, half, rows_sl])
      plsc.subcore_barrier()

  return edge_w, edge_agg


def _edge_phase(rec_cat, q_cat, k_cat, v0_cat, v1_cat):
  ew, ea = _sc_kernels()
  w_e, den = ew(rec_cat, q_cat, k_cat)
  wv = ea(rec_cat, v0_cat, v1_cat, w_e)
  return w_e, den, wv


def _out_body(wv0_ref, wv1_ref, den_ref, xp_ref, xa_ref, w_ref, b_ref, s_ref,
              o_ref):
    t = pl.program_id(0)
    x = jnp.where(t == 0, xp_ref[...], xa_ref[...])
    wv = jnp.concatenate([wv0_ref[0, 0], wv1_ref[0, 0]], axis=1)
    den = den_ref[0]
    parts = []
    for h in range(H):
        parts.append(wv[:, h * 32:(h + 1) * 32] / (den[:, h:h + 1] + 1e-16))
    agg = jnp.concatenate(parts, axis=1)
    g = 0.5 * agg * (1.0 + lax.erf(agg * (1.0 / math.sqrt(2.0))))
    y = jnp.dot(g, w_ref[0], preferred_element_type=jnp.float32) + b_ref[0]
    a = s_ref[0, 0, 0]
    o_ref[0] = a * y + (1.0 - a) * x


def _epilogue(wv, den, x_author, x_paper, w_st, b_st, sig_st):
    nb = 10
    rb = N_NODE // nb
    return pl.pallas_call(
        _out_body,
        grid=(2, nb),
        in_specs=[
            pl.BlockSpec((1, 1, rb, 128), lambda t, i: (t, 0, i, 0)),
            pl.BlockSpec((1, 1, rb, 128), lambda t, i: (t, 1, i, 0)),
            pl.BlockSpec((1, rb, 16), lambda t, i: (t, i, 0)),
            pl.BlockSpec((rb, D_IN), lambda t, i: (i, 0)),
            pl.BlockSpec((rb, D_IN), lambda t, i: (i, 0)),
            pl.BlockSpec((1, D_OUT, D_OUT), lambda t, i: (t, 0, 0)),
            pl.BlockSpec((1, 1, D_OUT), lambda t, i: (t, 0, 0)),
            pl.BlockSpec((1, 1, 1), lambda t, i: (t, 0, 0)),
        ],
        out_specs=pl.BlockSpec((1, rb, D_OUT), lambda t, i: (t, i, 0)),
        out_shape=jax.ShapeDtypeStruct((2, N_NODE, D_OUT), jnp.float32),
    )(wv, wv, den, x_paper, x_author, w_st, b_st, sig_st)


def kernel(x_author, x_paper, edge_index_writes, edge_index_rev,
           W_kqv_author, b_kqv_author, W_kqv_paper, b_kqv_paper,
           W_k_rel, W_v_rel,
           W_out_author, b_out_author, W_out_paper, b_out_paper,
           skip_author, skip_paper, p_rel_writes, p_rel_rev):
    scale = 1.0 / math.sqrt(D)
    hidx = jnp.arange(H) * 2

    def fold(W_kqv, b_kqv, et, p_rel):
        Wk = W_kqv[:, 0:256].reshape(D_IN, H, D)
        Wq = W_kqv[:, 256:512].reshape(D_IN, H, D)
        Wv = W_kqv[:, 512:768].reshape(D_IN, H, D)
        bk = b_kqv[0:256].reshape(H, D)
        bq = b_kqv[256:512].reshape(H, D)
        bv = b_kqv[512:768].reshape(H, D)
        Rk = W_k_rel[hidx + et]
        Rv = W_v_rel[hidx + et]
        qs = (p_rel[0] * scale)[None, :, None]
        Wq2 = (Wq * qs).reshape(D_IN, 256)
        bq2 = (bq * qs[0]).reshape(1, 256)
        Wk2 = jnp.einsum('ihd,hdo->iho', Wk, Rk).reshape(D_IN, 256)
        bk2 = jnp.einsum('hd,hdo->ho', bk, Rk).reshape(256)
        Wv2 = jnp.einsum('ihd,hdo->iho', Wv, Rv).reshape(D_IN, 256)
        bv2 = jnp.einsum('hd,hdo->ho', bv, Rv).reshape(256)
        Wkv = jnp.concatenate([Wk2, Wv2], axis=1)
        bkv = jnp.concatenate([bk2, bv2])[None, :]
        return Wq2, bq2, Wkv, bkv

    Wq_a, bq_a, Wkv_a, bkv_a = fold(W_kqv_author, b_kqv_author, 0, p_rel_rev)
    Wq_p, bq_p, Wkv_p, bkv_p = fold(W_kqv_paper, b_kqv_paper, 1, p_rel_writes)

    wq_st = jnp.stack([Wq_p, Wq_a])
    bq_st = jnp.stack([bq_p, bq_a])
    wkv_st = jnp.stack([Wkv_a, Wkv_p])
    bkv_st = jnp.stack([bkv_a, bkv_p])

    q_cat, k_cat, v0_cat, v1_cat = _qkv(x_author, x_paper,
                                        wq_st, bq_st, wkv_st, bkv_st)

    rec_cat = jnp.concatenate(
        [edge_index_writes.T, edge_index_rev.T], axis=0)

    _, den, wv = _edge_phase(rec_cat, q_cat, k_cat, v0_cat, v1_cat)

    w_out_st = jnp.stack([W_out_paper, W_out_author])
    b_out_st = jnp.stack([b_out_paper[None, :], b_out_author[None, :]])
    sig_st = jnp.stack([jax.nn.sigmoid(skip_paper),
                        jax.nn.sigmoid(skip_author)]).reshape(2, 1, 1)
    out = _epilogue(wv, den, x_author, x_paper, w_out_st, b_out_st, sig_st)
    return out[1], out[0]
